# Initial kernel scaffold; baseline (speedup 1.0000x reference)
import jax
import jax.numpy as jnp
from jax.experimental import pallas as pl


def kernel(arm_loc_data, arm_conf_data, odm_loc_data, odm_conf_data, prior_data):
    raise NotImplementedError("write your pallas kernel here")


# speedup vs baseline: 13.5637x; 13.5637x over previous
import jax
import jax.numpy as jnp
from jax import lax
from jax.experimental import pallas as pl
from jax.experimental.pallas import tpu as pltpu

_C = 21
_TOPK = 400
_NMS_T = 0.45
_ARM_VAR = (0.1, 0.2)
_ODM_VAR = (0.1, 0.2)
_POS_T = 0.01
_CONF_T = 0.01
_LANES = 128


def _sm(x):
    m = x.max(axis=-1, keepdims=True)
    e = jnp.exp(x - m)
    return e / e.sum(axis=-1, keepdims=True)


def _nms_body(score_ref, x1_ref, y1_ref, x2_ref, y2_ref, pidx_ref,
              outs_ref, ox1_ref, oy1_ref, ox2_ref, oy2_ref,
              act_ref, area_ref):
    score0 = score_ref[...]
    x1s = x1_ref[...]
    y1s = y1_ref[...]
    x2s = x2_ref[...]
    y2s = y2_ref[...]
    act_ref[...] = jnp.where(score0 > 0.0, 1.0, 0.0)
    area_ref[...] = (x2s - x1s) * (y2s - y1s)
    zeros = jnp.zeros_like(score0)
    outs_ref[...] = zeros
    ox1_ref[...] = zeros
    oy1_ref[...] = zeros
    ox2_ref[...] = zeros
    oy2_ref[...] = zeros

    def body(t, carry):
        act = act_ref[...] > 0.5
        score = score_ref[...]
        x1 = x1_ref[...]
        y1 = y1_ref[...]
        x2 = x2_ref[...]
        y2 = y2_ref[...]
        area = area_ref[...]
        pidx = pidx_ref[...]
        ms = jnp.where(act, score, -1.0)
        m = jnp.max(ms, axis=0, keepdims=True)
        has = m > 0.0
        cand = act & (score == m)
        tie = jnp.where(cand, pidx, -1.0)
        pmax = jnp.max(tie, axis=0, keepdims=True)
        onehot = cand & (pidx == pmax)
        oh = jnp.where(onehot, 1.0, 0.0)
        px1 = jnp.sum(oh * x1, axis=0, keepdims=True)
        py1 = jnp.sum(oh * y1, axis=0, keepdims=True)
        px2 = jnp.sum(oh * x2, axis=0, keepdims=True)
        py2 = jnp.sum(oh * y2, axis=0, keepdims=True)
        parea = jnp.sum(oh * area, axis=0, keepdims=True)
        xx1 = jnp.maximum(x1, px1)
        yy1 = jnp.maximum(y1, py1)
        xx2 = jnp.minimum(x2, px2)
        yy2 = jnp.minimum(y2, py2)
        w = jnp.clip(xx2 - xx1, 0.0, None)
        h = jnp.clip(yy2 - yy1, 0.0, None)
        inter = w * h
        union = (area - inter) + parea
        iou = inter / union
        keep = act & (iou <= _NMS_T) & jnp.logical_not(onehot)
        act_ref[...] = jnp.where(keep, 1.0, 0.0)
        outs_ref[pl.ds(t, 1), :] = jnp.where(has, m, 0.0)
        ox1_ref[pl.ds(t, 1), :] = jnp.where(has, px1, 0.0)
        oy1_ref[pl.ds(t, 1), :] = jnp.where(has, py1, 0.0)
        ox2_ref[pl.ds(t, 1), :] = jnp.where(has, px2, 0.0)
        oy2_ref[pl.ds(t, 1), :] = jnp.where(has, py2, 0.0)
        return carry

    lax.fori_loop(0, _TOPK, body, 0)


def _run_nms(score_t, x1_t, y1_t, x2_t, y2_t, pidx_t):
    shp = jax.ShapeDtypeStruct((_TOPK, _LANES), jnp.float32)
    return pl.pallas_call(
        _nms_body,
        out_shape=[shp] * 5,
        scratch_shapes=[pltpu.VMEM((_TOPK, _LANES), jnp.float32)] * 2,
    )(score_t, x1_t, y1_t, x2_t, y2_t, pidx_t)


def kernel(arm_loc_data, arm_conf_data, odm_loc_data, odm_conf_data, prior_data):
    num, P, _ = arm_loc_data.shape
    nc = _C - 1
    npb = num * nc

    arm_score = _sm(arm_conf_data)
    score = _sm(odm_conf_data)
    centers = prior_data[None, :, :2] + arm_loc_data[:, :, :2] * _ARM_VAR[0] * prior_data[None, :, 2:]
    wh = prior_data[None, :, 2:] * jnp.exp(arm_loc_data[:, :, 2:] * _ARM_VAR[1])
    refined = jnp.concatenate([centers, wh], axis=2)
    xy = refined[..., :2] + odm_loc_data[..., :2] * _ODM_VAR[0] * refined[..., 2:]
    bwh = refined[..., 2:] * jnp.exp(odm_loc_data[..., 2:] * _ODM_VAR[1])
    x1y1 = xy - bwh / 2.0
    x2y2 = bwh + x1y1
    all_boxes = jnp.concatenate([x1y1, x2y2], axis=-1)

    flag = arm_score[:, :, 1] > _POS_T
    cls_scores = jnp.transpose(score, (0, 2, 1))[:, 1:, :]
    valid = flag[:, None, :] & (cls_scores > _CONF_T)
    masked = jnp.where(valid, cls_scores, -1.0).reshape(npb, P)

    vals, idxs = lax.top_k(masked, _TOPK)
    bidx = (jnp.arange(npb) // nc)[:, None]
    cboxes = all_boxes[bidx, idxs]
    pidx = idxs.astype(jnp.float32)

    def plane(a, pad):
        a = jnp.pad(a, ((0, _LANES - npb), (0, 0)), constant_values=pad)
        return a.T

    score_t = plane(vals, -1.0)
    x1_t = plane(cboxes[..., 0], 0.0)
    y1_t = plane(cboxes[..., 1], 0.0)
    x2_t = plane(cboxes[..., 2], 0.0)
    y2_t = plane(cboxes[..., 3], 0.0)
    pidx_t = plane(pidx, 0.0)

    outs, ox1, oy1, ox2, oy2 = _run_nms(score_t, x1_t, y1_t, x2_t, y2_t, pidx_t)

    sel_s = outs.T[:npb]
    dets = jnp.stack([sel_s, ox1.T[:npb], oy1.T[:npb], ox2.T[:npb], oy2.T[:npb]], axis=-1)
    dets = dets.reshape(num, nc, _TOPK, 5)
    output = jnp.zeros((num, _C, _TOPK, 5), dtype=jnp.float32)
    output = output.at[:, 1:].set(dets)
    return output
